# baseline (speedup 1.0000x reference)
import functools

import jax
import jax.numpy as jnp
from jax import lax
from jax.experimental import pallas as pl
from jax.experimental.pallas import tpu as pltpu
from jax.experimental.pallas import tpu_sc as plsc

N_TOK = 16 * 2048
D = 128
M = 1000
MP = 1024
TILE = 4096
T = N_TOK // TILE

NC = 2
NS = 16
CHUNK = 128
ROWS_PER_WORKER = N_TOK // (NC * NS)


def _stage_a(q_ref, k_ref, qw_ref, gi_ref, colmax_out_ref, kb_ref, colmax_ref):
    t = pl.program_id(0)

    @pl.when(t == 0)
    def _init():
        colmax_ref[...] = jnp.full((1, MP), -1e30, jnp.float32)
        kb_ref[...] = k_ref[...].astype(jnp.bfloat16)

    q = q_ref[...]
    ones = jnp.ones((D, D), jnp.bfloat16)
    ss = jnp.dot((q * q).astype(jnp.bfloat16), ones, preferred_element_type=jnp.float32)
    qn = q * lax.rsqrt(jnp.maximum(ss, 1e-24))
    qb = qn.astype(jnp.bfloat16)
    s = jnp.dot(qb, kb_ref[...].T, preferred_element_type=jnp.float32)
    col_ids = lax.broadcasted_iota(jnp.int32, (TILE, MP), 1)
    s = jnp.where(col_ids < M, s, -1e30)
    rowmax = jnp.max(s, axis=1, keepdims=True)
    gi = jnp.min(jnp.where(s == rowmax, col_ids, MP), axis=1, keepdims=True)
    colmax_ref[...] = jnp.maximum(colmax_ref[...], jnp.max(s, axis=0, keepdims=True))
    qw_ref[...] = qn * jnp.exp(rowmax)
    lane = lax.broadcasted_iota(jnp.int32, (TILE, T), 1)
    gi_ref[...] = jnp.where(lane == t, gi, gi_ref[...])

    @pl.when(t == T - 1)
    def _finish():
        colmax_out_ref[...] = jnp.broadcast_to(colmax_ref[...], (8, MP))


def _sc_scatter_body(qw_hbm, gi_hbm, zeros_hbm, out_hbm, idx_v, rows_v, table_sh):
    c = lax.axis_index("c")
    s = lax.axis_index("s")
    base = (c * NS + s) * ROWS_PER_WORKER
    rows_per_sub = MP // NS
    pltpu.sync_copy(
        zeros_hbm.at[pl.ds(s * rows_per_sub, rows_per_sub)],
        table_sh.at[pl.ds(s * rows_per_sub, rows_per_sub)],
    )
    plsc.subcore_barrier()
    for j in range(ROWS_PER_WORKER // CHUNK):
        off = base + j * CHUNK
        pltpu.sync_copy(gi_hbm.at[pl.ds(off, CHUNK)], idx_v)
        pltpu.sync_copy(qw_hbm.at[pl.ds(off, CHUNK)], rows_v)
        pltpu.sync_copy(rows_v, table_sh.at[idx_v], add=True)
    plsc.subcore_barrier()
    pltpu.sync_copy(
        table_sh.at[pl.ds(s * rows_per_sub, rows_per_sub)],
        out_hbm.at[c, pl.ds(s * rows_per_sub, rows_per_sub)],
    )


def _stage_c(tabs_ref, colmax_ref, k_ref, out_ref):
    tab = tabs_ref[0] + tabs_ref[1]
    cm = jnp.transpose(jnp.exp(-colmax_ref[0:1, :]))
    upd = 1e-05 * cm * tab + k_ref[...]
    nrm = jnp.sum(upd * upd, axis=1, keepdims=True)
    out_ref[...] = (upd * lax.rsqrt(jnp.maximum(nrm, 1e-24)))[:M]


@jax.jit
def kernel(query, keys):
    q2 = query.reshape(N_TOK, D)
    kp = jnp.pad(keys, ((0, MP - M), (0, 0)))

    qw, gi_cols, colmax = pl.pallas_call(
        _stage_a,
        grid=(T,),
        in_specs=[
            pl.BlockSpec((TILE, D), lambda t: (t, 0)),
            pl.BlockSpec((MP, D), lambda t: (0, 0)),
        ],
        out_specs=[
            pl.BlockSpec((TILE, D), lambda t: (t, 0)),
            pl.BlockSpec((TILE, T), lambda t: (0, 0)),
            pl.BlockSpec((8, MP), lambda t: (0, 0)),
        ],
        out_shape=[
            jax.ShapeDtypeStruct((N_TOK, D), jnp.float32),
            jax.ShapeDtypeStruct((TILE, T), jnp.int32),
            jax.ShapeDtypeStruct((8, MP), jnp.float32),
        ],
        scratch_shapes=[
            pltpu.VMEM((MP, D), jnp.bfloat16),
            pltpu.VMEM((1, MP), jnp.float32),
        ],
    )(q2, kp)

    gi_flat = gi_cols.T.reshape(N_TOK)
    zeros = jnp.zeros((MP, D), jnp.float32)

    sc_scatter = functools.partial(
        pl.kernel,
        mesh=plsc.VectorSubcoreMesh(core_axis_name="c", subcore_axis_name="s"),
        out_type=jax.ShapeDtypeStruct((NC, MP, D), jnp.float32),
        scratch_types=[
            pltpu.VMEM((CHUNK,), jnp.int32),
            pltpu.VMEM((CHUNK, D), jnp.float32),
            pltpu.VMEM_SHARED((MP, D), jnp.float32),
        ],
    )(_sc_scatter_body)
    tabs = sc_scatter(qw, gi_flat, zeros)

    return pl.pallas_call(
        _stage_c,
        in_specs=[
            pl.BlockSpec((NC, MP, D), lambda: (0, 0, 0)),
            pl.BlockSpec((8, MP), lambda: (0, 0)),
            pl.BlockSpec((MP, D), lambda: (0, 0)),
        ],
        out_specs=pl.BlockSpec((M, D), lambda: (0, 0)),
        out_shape=jax.ShapeDtypeStruct((M, D), jnp.float32),
    )(tabs, colmax, kp)


# speedup vs baseline: 2.3650x; 2.3650x over previous
import jax
import jax.numpy as jnp
from jax.experimental import pallas as pl
from jax.experimental.pallas import tpu as pltpu

N_TOK = 16 * 2048
D = 128
M = 1000
MP = 1024
TILE = 4096
T = N_TOK // TILE


def _body(q_ref, k_ref, out_ref, kb_ref, colmax_ref, acc_ref):
    t = pl.program_id(0)

    @pl.when(t == 0)
    def _init():
        colmax_ref[...] = jnp.full((1, MP), -1e30, jnp.float32)
        acc_ref[...] = jnp.zeros((D, MP), jnp.float32)
        kb_ref[...] = k_ref[...].astype(jnp.float8_e4m3fn)

    q = q_ref[...]
    ones = jnp.ones((D, D), jnp.bfloat16)
    ss = jnp.dot((q * q).astype(jnp.bfloat16), ones, preferred_element_type=jnp.float32)
    qn = q * jax.lax.rsqrt(jnp.maximum(ss, 1e-24))
    qb = qn.astype(jnp.float8_e4m3fn)
    s = jnp.dot(qb, kb_ref[...].T, preferred_element_type=jnp.float32)
    col_ids = jax.lax.broadcasted_iota(jnp.int32, (TILE, MP), 1)
    sb = jnp.where(col_ids < M, s.astype(jnp.bfloat16), jnp.bfloat16(-1e30))
    rowmax = jnp.max(sb, axis=1, keepdims=True)
    colmax_ref[...] = jnp.maximum(
        colmax_ref[...], jnp.max(sb, axis=0, keepdims=True).astype(jnp.float32)
    )

    onehot = jnp.where(sb == rowmax, jnp.bfloat16(1), jnp.bfloat16(0))
    qw = (qn * jnp.exp(rowmax.astype(jnp.float32))).astype(jnp.bfloat16)
    acc_ref[...] += jax.lax.dot_general(
        qw, onehot, (((0,), (0,)), ((), ())), preferred_element_type=jnp.float32
    )

    @pl.when(t == T - 1)
    def _finish():
        ut = 1e-05 * jnp.exp(-colmax_ref[...]) * acc_ref[...]
        upd = jnp.transpose(ut) + k_ref[...]
        nrm = jnp.sum(upd * upd, axis=1, keepdims=True)
        out_ref[...] = (upd * jax.lax.rsqrt(jnp.maximum(nrm, 1e-24)))[:M]


@jax.jit
def kernel(query, keys):
    q2 = query.reshape(N_TOK, D)
    kp = jnp.pad(keys, ((0, MP - M), (0, 0)))
    return pl.pallas_call(
        _body,
        grid=(T,),
        in_specs=[
            pl.BlockSpec((TILE, D), lambda t: (t, 0)),
            pl.BlockSpec((MP, D), lambda t: (0, 0)),
        ],
        out_specs=pl.BlockSpec((M, D), lambda t: (0, 0)),
        out_shape=jax.ShapeDtypeStruct((M, D), jnp.float32),
        scratch_shapes=[
            pltpu.VMEM((MP, D), jnp.float8_e4m3fn),
            pltpu.VMEM((1, MP), jnp.float32),
            pltpu.VMEM((D, MP), jnp.float32),
        ],
    )(q2, kp)


# speedup vs baseline: 2.6245x; 1.1097x over previous
import jax
import jax.numpy as jnp
from jax.experimental import pallas as pl
from jax.experimental.pallas import tpu as pltpu

N_TOK = 16 * 2048
D = 128
M = 1000
MP = 1024
TILE = 4096
T = N_TOK // TILE


def _body(q_ref, k_ref, out_ref, kb_ref, colmax_ref, acc_ref):
    t = pl.program_id(0)

    @pl.when(t == 0)
    def _init():
        colmax_ref[...] = jnp.full((1, MP), -1e30, jnp.float32)
        acc_ref[...] = jnp.zeros((D, MP), jnp.float32)
        kb_ref[...] = k_ref[...].astype(jnp.bfloat16)

    q = q_ref[...]
    ones = jnp.ones((D, D), jnp.bfloat16)
    ss = jnp.dot((q * q).astype(jnp.bfloat16), ones, preferred_element_type=jnp.float32)
    qn = q * jax.lax.rsqrt(jnp.maximum(ss, 1e-24))
    qb = qn.astype(jnp.bfloat16)
    s = jnp.dot(qb, kb_ref[...].T, preferred_element_type=jnp.float32)
    sb = s.astype(jnp.bfloat16)
    rowmax = jnp.max(sb, axis=1, keepdims=True)
    colmax_ref[...] = jnp.maximum(
        colmax_ref[...], jnp.max(sb, axis=0, keepdims=True).astype(jnp.float32)
    )

    onehot = jnp.where(sb == rowmax, jnp.bfloat16(1), jnp.bfloat16(0))
    qw = (qn * jnp.exp(rowmax.astype(jnp.float32))).astype(jnp.bfloat16)
    acc_ref[...] += jax.lax.dot_general(
        qw, onehot, (((0,), (0,)), ((), ())), preferred_element_type=jnp.float32
    )

    @pl.when(t == T - 1)
    def _finish():
        ut = 1e-05 * jnp.exp(-colmax_ref[...]) * acc_ref[...]
        upd = jnp.transpose(ut) + k_ref[...]
        nrm = jnp.sum(upd * upd, axis=1, keepdims=True)
        out_ref[...] = (upd * jax.lax.rsqrt(jnp.maximum(nrm, 1e-24)))[:M]


@jax.jit
def kernel(query, keys):
    q2 = query.reshape(N_TOK, D)
    kp = jnp.pad(keys, ((0, MP - M), (0, 0)))
    return pl.pallas_call(
        _body,
        grid=(T,),
        in_specs=[
            pl.BlockSpec((TILE, D), lambda t: (t, 0)),
            pl.BlockSpec((MP, D), lambda t: (0, 0)),
        ],
        out_specs=pl.BlockSpec((M, D), lambda t: (0, 0)),
        out_shape=jax.ShapeDtypeStruct((M, D), jnp.float32),
        scratch_shapes=[
            pltpu.VMEM((MP, D), jnp.bfloat16),
            pltpu.VMEM((1, MP), jnp.float32),
            pltpu.VMEM((D, MP), jnp.float32),
        ],
    )(q2, kp)
